# baseline (speedup 1.0000x reference)
import functools

import jax
import jax.numpy as jnp
from jax import lax
from jax.experimental import pallas as pl
from jax.experimental.pallas import tpu as pltpu
from jax.experimental.pallas import tpu_sc as plsc

NUM_VALUES = 1000000
EMBED_DIM = 16
BATCH = 16384

_INFO = plsc.get_sparse_core_info()
_NC, _NS, _L = _INFO.num_cores, _INFO.num_subcores, _INFO.num_lanes
_NW = _NC * _NS
_B_PER_W = BATCH // _NW
_CHUNK = 128
_N_CHUNKS = _B_PER_W // _CHUNK


def _flatten_kernel(tab_t_ref, *rest):
    out_refs, sem = rest[:EMBED_DIM], rest[EMBED_DIM]
    copies = [
        pltpu.make_async_copy(tab_t_ref.at[d], out_refs[d], sem)
        for d in range(EMBED_DIM)
    ]
    for cp in copies:
        cp.start()
    for cp in copies:
        cp.wait()


def _flatten(tab_t):
    return pl.pallas_call(
        _flatten_kernel,
        in_specs=[pl.BlockSpec(memory_space=pl.ANY)],
        out_specs=[pl.BlockSpec(memory_space=pl.ANY)] * EMBED_DIM,
        out_shape=[
            jax.ShapeDtypeStruct((NUM_VALUES,), jnp.float32)
        ] * EMBED_DIM,
        scratch_shapes=[pltpu.SemaphoreType.DMA],
    )(tab_t)


def _make_gather():
    mesh = plsc.VectorSubcoreMesh(core_axis_name="c", subcore_axis_name="s")

    @functools.partial(
        pl.kernel,
        mesh=mesh,
        out_type=jax.ShapeDtypeStruct((EMBED_DIM, BATCH), jnp.float32),
        scratch_types=[
            pltpu.VMEM((_B_PER_W,), jnp.int32),
            pltpu.VMEM((EMBED_DIM, _B_PER_W), jnp.float32),
            pltpu.SemaphoreType.DMA,
        ],
    )
    def gather_kernel(idx_hbm, *rest):
        plane_hbm = rest[:EMBED_DIM]
        out_hbm, idx_v, cols_v, sem = rest[EMBED_DIM:]
        wid = lax.axis_index("s") * _NC + lax.axis_index("c")
        base = wid * _B_PER_W

        pltpu.sync_copy(idx_hbm.at[pl.ds(base, _B_PER_W)], idx_v)

        for k in range(_B_PER_W // _L):
            v = idx_v[pl.ds(k * _L, _L)]
            idx_v[pl.ds(k * _L, _L)] = jnp.where(v >= NUM_VALUES, 0, v)

        copies = []
        for d in range(EMBED_DIM):
            for j in range(_N_CHUNKS):
                copies.append(
                    pltpu.async_copy(
                        plane_hbm[d].at[idx_v.at[pl.ds(j * _CHUNK, _CHUNK)]],
                        cols_v.at[d, pl.ds(j * _CHUNK, _CHUNK)],
                        sem,
                    )
                )
        for cp in copies:
            cp.wait()

        pltpu.sync_copy(cols_v, out_hbm.at[:, pl.ds(base, _B_PER_W)])

    return gather_kernel


_GATHER = _make_gather()


def kernel(categorical_column, table):
    idx = categorical_column.astype(jnp.int32)
    tab_t = table.T
    planes = _flatten(tab_t)
    out_t = _GATHER(idx, *planes)
    return out_t.T


# speedup vs baseline: 26.2651x; 26.2651x over previous
import functools

import jax
import jax.numpy as jnp
from jax import lax
from jax.experimental import pallas as pl
from jax.experimental.pallas import tpu as pltpu
from jax.experimental.pallas import tpu_sc as plsc

NUM_VALUES = 1000000
EMBED_DIM = 16
BATCH = 16384

_INFO = plsc.get_sparse_core_info()
_NC, _NS, _L = _INFO.num_cores, _INFO.num_subcores, _INFO.num_lanes
_NW = _NC * _NS
_B_PER_W = BATCH // _NW
_CHUNK = 128
_N_CHUNKS = _B_PER_W // _CHUNK


_COL_BLOCK = 65536
_N_COL_BLOCKS = -(-NUM_VALUES // _COL_BLOCK)
_PLANE = _N_COL_BLOCKS * _COL_BLOCK


def _flatten_kernel(tab_t_ref, flat_ref, sem):
    j = pl.program_id(0)
    copies = [
        pltpu.make_async_copy(
            tab_t_ref.at[d],
            flat_ref.at[pl.ds(d * _PLANE + j * _COL_BLOCK, _COL_BLOCK)],
            sem,
        )
        for d in range(EMBED_DIM)
    ]
    for cp in copies:
        cp.start()
    for cp in copies:
        cp.wait()


def _flatten(tab_t):
    return pl.pallas_call(
        _flatten_kernel,
        grid=(_N_COL_BLOCKS,),
        in_specs=[
            pl.BlockSpec((EMBED_DIM, _COL_BLOCK), lambda j: (0, j)),
        ],
        out_specs=pl.BlockSpec(memory_space=pl.ANY),
        out_shape=jax.ShapeDtypeStruct((EMBED_DIM * _PLANE,), jnp.float32),
        scratch_shapes=[pltpu.SemaphoreType.DMA],
    )(tab_t)


def _make_gather():
    mesh = plsc.VectorSubcoreMesh(core_axis_name="c", subcore_axis_name="s")

    @functools.partial(
        pl.kernel,
        mesh=mesh,
        out_type=jax.ShapeDtypeStruct((EMBED_DIM, BATCH), jnp.float32),
        scratch_types=[
            pltpu.VMEM((_B_PER_W,), jnp.int32),
            pltpu.VMEM((EMBED_DIM, _B_PER_W), jnp.float32),
            pltpu.SemaphoreType.DMA,
        ],
    )
    def gather_kernel(idx_hbm, flat_hbm, out_hbm, idx_v, cols_v, sem):
        wid = lax.axis_index("s") * _NC + lax.axis_index("c")
        base = wid * _B_PER_W

        pltpu.sync_copy(idx_hbm.at[pl.ds(base, _B_PER_W)], idx_v)

        for k in range(_B_PER_W // _L):
            v = idx_v[pl.ds(k * _L, _L)]
            idx_v[pl.ds(k * _L, _L)] = jnp.where(v >= NUM_VALUES, 0, v)

        copies = []
        for d in range(EMBED_DIM):
            plane = flat_hbm.at[pl.ds(d * _PLANE, _PLANE)]
            for j in range(_N_CHUNKS):
                copies.append(
                    pltpu.async_copy(
                        plane.at[idx_v.at[pl.ds(j * _CHUNK, _CHUNK)]],
                        cols_v.at[d, pl.ds(j * _CHUNK, _CHUNK)],
                        sem,
                    )
                )
        for cp in copies:
            cp.wait()

        pltpu.sync_copy(cols_v, out_hbm.at[:, pl.ds(base, _B_PER_W)])

    return gather_kernel


_GATHER = _make_gather()


def kernel(categorical_column, table):
    idx = categorical_column.astype(jnp.int32)
    tab_t = table.T
    flat = _flatten(tab_t)
    out_t = _GATHER(idx, flat)
    return out_t.T


# speedup vs baseline: 28.4295x; 1.0824x over previous
import functools

import jax
import jax.numpy as jnp
from jax import lax
from jax.experimental import pallas as pl
from jax.experimental.pallas import tpu as pltpu
from jax.experimental.pallas import tpu_sc as plsc

NUM_VALUES = 1000000
EMBED_DIM = 16
BATCH = 16384

_INFO = plsc.get_sparse_core_info()
_NC, _NS, _L = _INFO.num_cores, _INFO.num_subcores, _INFO.num_lanes
_NW = _NC * _NS
_B_PER_W = BATCH // _NW
_CHUNK = 128
_N_CHUNKS = _B_PER_W // _CHUNK


_COL_BLOCK = 131072
_N_COL_BLOCKS = -(-NUM_VALUES // _COL_BLOCK)
_PLANE = _N_COL_BLOCKS * _COL_BLOCK


def _flatten_kernel(tab_t_ref, flat_ref, sem):
    j = pl.program_id(0)
    copies = [
        pltpu.make_async_copy(
            tab_t_ref.at[d],
            flat_ref.at[pl.ds(d * _PLANE + j * _COL_BLOCK, _COL_BLOCK)],
            sem,
        )
        for d in range(EMBED_DIM)
    ]
    for cp in copies:
        cp.start()
    for cp in copies:
        cp.wait()


def _flatten(tab_t):
    return pl.pallas_call(
        _flatten_kernel,
        grid=(_N_COL_BLOCKS,),
        in_specs=[
            pl.BlockSpec((EMBED_DIM, _COL_BLOCK), lambda j: (0, j)),
        ],
        out_specs=pl.BlockSpec(memory_space=pl.ANY),
        out_shape=jax.ShapeDtypeStruct((EMBED_DIM * _PLANE,), jnp.float32),
        scratch_shapes=[pltpu.SemaphoreType.DMA],
    )(tab_t)


def _make_gather():
    mesh = plsc.VectorSubcoreMesh(core_axis_name="c", subcore_axis_name="s")

    @functools.partial(
        pl.kernel,
        mesh=mesh,
        out_type=jax.ShapeDtypeStruct((EMBED_DIM, BATCH), jnp.float32),
        scratch_types=[
            pltpu.VMEM((_B_PER_W,), jnp.int32),
            pltpu.VMEM((EMBED_DIM, _B_PER_W), jnp.float32),
            pltpu.SemaphoreType.DMA,
        ],
    )
    def gather_kernel(idx_hbm, flat_hbm, out_hbm, idx_v, cols_v, sem):
        wid = lax.axis_index("s") * _NC + lax.axis_index("c")
        base = wid * _B_PER_W

        pltpu.sync_copy(idx_hbm.at[pl.ds(base, _B_PER_W)], idx_v)

        for k in range(_B_PER_W // _L):
            v = idx_v[pl.ds(k * _L, _L)]
            idx_v[pl.ds(k * _L, _L)] = jnp.where(v >= NUM_VALUES, 0, v)

        copies = []
        for d in range(EMBED_DIM):
            plane = flat_hbm.at[pl.ds(d * _PLANE, _PLANE)]
            for j in range(_N_CHUNKS):
                copies.append(
                    pltpu.async_copy(
                        plane.at[idx_v.at[pl.ds(j * _CHUNK, _CHUNK)]],
                        cols_v.at[d, pl.ds(j * _CHUNK, _CHUNK)],
                        sem,
                    )
                )
        for cp in copies:
            cp.wait()

        pltpu.sync_copy(cols_v, out_hbm.at[:, pl.ds(base, _B_PER_W)])

    return gather_kernel


_GATHER = _make_gather()


def kernel(categorical_column, table):
    idx = categorical_column.astype(jnp.int32)
    tab_t = table.T
    flat = _flatten(tab_t)
    out_t = _GATHER(idx, flat)
    return out_t.T


# speedup vs baseline: 28.9029x; 1.0167x over previous
import functools

import jax
import jax.numpy as jnp
from jax import lax
from jax.experimental import pallas as pl
from jax.experimental.pallas import tpu as pltpu
from jax.experimental.pallas import tpu_sc as plsc

NUM_VALUES = 1000000
EMBED_DIM = 16
BATCH = 16384

_INFO = plsc.get_sparse_core_info()
_NC, _NS, _L = _INFO.num_cores, _INFO.num_subcores, _INFO.num_lanes
_NW = _NC * _NS
_B_PER_W = BATCH // _NW
_CHUNK = 128
_N_CHUNKS = _B_PER_W // _CHUNK


_COL_BLOCK = 262144
_N_COL_BLOCKS = -(-NUM_VALUES // _COL_BLOCK)
_PLANE = _N_COL_BLOCKS * _COL_BLOCK


def _flatten_kernel(tab_t_ref, flat_ref, sem):
    j = pl.program_id(0)
    copies = [
        pltpu.make_async_copy(
            tab_t_ref.at[d],
            flat_ref.at[pl.ds(d * _PLANE + j * _COL_BLOCK, _COL_BLOCK)],
            sem,
        )
        for d in range(EMBED_DIM)
    ]
    for cp in copies:
        cp.start()
    for cp in copies:
        cp.wait()


def _flatten(tab_t):
    return pl.pallas_call(
        _flatten_kernel,
        grid=(_N_COL_BLOCKS,),
        in_specs=[
            pl.BlockSpec((EMBED_DIM, _COL_BLOCK), lambda j: (0, j)),
        ],
        out_specs=pl.BlockSpec(memory_space=pl.ANY),
        out_shape=jax.ShapeDtypeStruct((EMBED_DIM * _PLANE,), jnp.float32),
        scratch_shapes=[pltpu.SemaphoreType.DMA],
    )(tab_t)


def _make_gather():
    mesh = plsc.VectorSubcoreMesh(core_axis_name="c", subcore_axis_name="s")

    @functools.partial(
        pl.kernel,
        mesh=mesh,
        out_type=jax.ShapeDtypeStruct((EMBED_DIM, BATCH), jnp.float32),
        scratch_types=[
            pltpu.VMEM((_B_PER_W,), jnp.int32),
            pltpu.VMEM((EMBED_DIM, _B_PER_W), jnp.float32),
            pltpu.SemaphoreType.DMA,
        ],
    )
    def gather_kernel(idx_hbm, flat_hbm, out_hbm, idx_v, cols_v, sem):
        wid = lax.axis_index("s") * _NC + lax.axis_index("c")
        base = wid * _B_PER_W

        pltpu.sync_copy(idx_hbm.at[pl.ds(base, _B_PER_W)], idx_v)

        for k in range(_B_PER_W // _L):
            v = idx_v[pl.ds(k * _L, _L)]
            idx_v[pl.ds(k * _L, _L)] = jnp.where(v >= NUM_VALUES, 0, v)

        copies = []
        for d in range(EMBED_DIM):
            plane = flat_hbm.at[pl.ds(d * _PLANE, _PLANE)]
            for j in range(_N_CHUNKS):
                copies.append(
                    pltpu.async_copy(
                        plane.at[idx_v.at[pl.ds(j * _CHUNK, _CHUNK)]],
                        cols_v.at[d, pl.ds(j * _CHUNK, _CHUNK)],
                        sem,
                    )
                )
        for cp in copies:
            cp.wait()

        pltpu.sync_copy(cols_v, out_hbm.at[:, pl.ds(base, _B_PER_W)])

    return gather_kernel


_GATHER = _make_gather()


def kernel(categorical_column, table):
    idx = categorical_column.astype(jnp.int32)
    tab_t = table.T
    flat = _flatten(tab_t)
    out_t = _GATHER(idx, flat)
    return out_t.T
